# baseline (speedup 1.0000x reference)
import dataclasses
import functools

import jax
import jax.numpy as jnp
from jax import lax
from jax.experimental import pallas as pl
from jax.experimental.pallas import tpu as pltpu
from jax.experimental.pallas import tpu_sc as plsc

N = 10000
D = 128
C = 64
E = 320000

NC = 2
NS = 16
NW = NC * NS
L = 16

CHUNK = 128
ITERS = 80
EPT = CHUNK * ITERS
EP = EPT * NW
TOTCH = EP // CHUNK

CH0 = 80
CH1 = 80
CHMAX = max(CH0, CH1)

SLICE = 632
NACC = NS * SLICE
NTRASH = 128
ACC_ROWS = NACC + NTRASH
DEG_LEN = NACC + NTRASH

_mesh = plsc.VectorSubcoreMesh(core_axis_name="c", subcore_axis_name="s")

_sc_params = pltpu.CompilerParams(needs_layout_passes=False,
                                  use_tc_tiling_on_sc=False)


def _deg_body(dst_hbm, out_hbm, idx_v, deg_v):
    cid = lax.axis_index("c")
    sid = lax.axis_index("s")
    g = cid * NS + sid
    zero = jnp.zeros((L,), jnp.float32)

    @pl.loop(0, DEG_LEN // L)
    def _(i):
        deg_v[pl.ds(i * L, L)] = zero

    pltpu.sync_copy(dst_hbm.at[pl.ds(g * EPT, EPT)], idx_v)
    ones = jnp.ones((L,), jnp.float32)

    @pl.loop(0, EPT // L)
    def _(i):
        idx = idx_v[pl.ds(i * L, L)]
        plsc.addupdate_scatter(deg_v, [idx], ones)

    pltpu.sync_copy(deg_v, out_hbm.at[g])


_deg_call = functools.partial(
    pl.kernel,
    out_type=jax.ShapeDtypeStruct((NW, DEG_LEN), jnp.float32),
    mesh=_mesh,
    compiler_params=_sc_params,
    scratch_types=[
        pltpu.VMEM((EPT,), jnp.int32),
        pltpu.VMEM((DEG_LEN,), jnp.float32),
    ],
)(_deg_body)


NBUF = 8


def _round_body(g_hbm, src_hbm, dst_hbm, out_hbm, src_v, dst_v,
                r0, r1, r2, r3, r4, r5, r6, r7, accum,
                gs0, gs1, gs2, gs3, gs4, gs5, gs6, gs7,
                ss0, ss1, ss2, ss3, ss4, ss5, ss6, ss7):
    rows = (r0, r1, r2, r3, r4, r5, r6, r7)
    gsem = (gs0, gs1, gs2, gs3, gs4, gs5, gs6, gs7)
    ssem = (ss0, ss1, ss2, ss3, ss4, ss5, ss6, ss7)
    cid = lax.axis_index("c")
    sid = lax.axis_index("s")
    zero = jnp.zeros((L,), jnp.float32)

    @pl.loop(0, CHUNK)
    def _(r):
        for k in range(C // L):
            r0[r, pl.ds(k * L, L)] = zero

    base = sid * SLICE
    nfull = SLICE // CHUNK
    rem = SLICE - nfull * CHUNK
    for k in range(nfull):
        pltpu.sync_copy(r0, accum.at[pl.ds(base + k * CHUNK, CHUNK)])
    if rem:
        pltpu.sync_copy(r0.at[pl.ds(0, rem)],
                        accum.at[pl.ds(base + nfull * CHUNK, rem)])
    plsc.subcore_barrier()

    def run_edges(start, cnt):
        pltpu.sync_copy(src_hbm.at[pl.ds(start, cnt)],
                        src_v.at[pl.ds(0, cnt)])
        pltpu.sync_copy(dst_hbm.at[pl.ds(start, cnt)],
                        dst_v.at[pl.ds(0, cnt)])

        for b in range(NBUF):
            pltpu.async_copy(g_hbm.at[src_v.at[b]], rows[b], gsem[b])

        @pl.loop(0, cnt, step=NBUF)
        def _(i0):
            for b in range(NBUF):
                i = i0 + b
                pltpu.make_async_copy(g_hbm.at[src_v.at[i]], rows[b],
                                      gsem[b]).wait()
                pltpu.async_copy(rows[b], accum.at[dst_v.at[i]], ssem[b],
                                 add=True)

                @pl.when(i0 + NBUF < cnt)
                def _():
                    pltpu.make_async_copy(rows[b], accum.at[dst_v.at[i]],
                                          ssem[b]).wait()
                    pltpu.async_copy(g_hbm.at[src_v.at[i + NBUF]], rows[b],
                                     gsem[b])

        for b in range(NBUF):
            i_last = cnt - NBUF + b
            pltpu.make_async_copy(rows[b], accum.at[dst_v.at[i_last]],
                                  ssem[b]).wait()

    if CH0:
        @pl.when(cid == 0)
        def _():
            run_edges(sid * CH0, CH0)

    if CH1:
        @pl.when(cid == 1)
        def _():
            run_edges(NS * CH0 + sid * CH1, CH1)

    plsc.subcore_barrier()
    pltpu.sync_copy(accum.at[pl.ds(sid * SLICE, SLICE)],
                    out_hbm.at[cid].at[pl.ds(sid * SLICE, SLICE)])


_round_call = functools.partial(
    pl.kernel,
    out_type=jax.ShapeDtypeStruct((NC, NACC, C), jnp.float32),
    mesh=_mesh,
    compiler_params=_sc_params,
    scratch_types=[
        pltpu.VMEM((CHMAX, CHUNK), jnp.int32),
        pltpu.VMEM((CHMAX, CHUNK), jnp.int32),
        pltpu.VMEM((CHUNK, C), jnp.float32),
        pltpu.VMEM((CHUNK, C), jnp.float32),
        pltpu.VMEM((CHUNK, C), jnp.float32),
        pltpu.VMEM((CHUNK, C), jnp.float32),
        pltpu.VMEM((CHUNK, C), jnp.float32),
        pltpu.VMEM((CHUNK, C), jnp.float32),
        pltpu.VMEM((CHUNK, C), jnp.float32),
        pltpu.VMEM((CHUNK, C), jnp.float32),
        pltpu.VMEM_SHARED((ACC_ROWS, C), jnp.float32),
        pltpu.SemaphoreType.DMA,
        pltpu.SemaphoreType.DMA,
        pltpu.SemaphoreType.DMA,
        pltpu.SemaphoreType.DMA,
        pltpu.SemaphoreType.DMA,
        pltpu.SemaphoreType.DMA,
        pltpu.SemaphoreType.DMA,
        pltpu.SemaphoreType.DMA,
        pltpu.SemaphoreType.DMA,
        pltpu.SemaphoreType.DMA,
        pltpu.SemaphoreType.DMA,
        pltpu.SemaphoreType.DMA,
        pltpu.SemaphoreType.DMA,
        pltpu.SemaphoreType.DMA,
        pltpu.SemaphoreType.DMA,
        pltpu.SemaphoreType.DMA,
    ],
)(_round_body)


def _matmul_body(x_ref, w_ref, y_ref):
    y_ref[...] = jnp.dot(x_ref[...], w_ref[...],
                         preferred_element_type=jnp.float32)


def _prep_body(degp_ref, y_ref, g0_ref, dinv_ref):
    deg = jnp.sum(degp_ref[...], axis=0)[:N] + 1.0
    dinv = lax.rsqrt(deg).reshape(N, 1)
    dinv_ref[...] = dinv
    g0_ref[...] = y_ref[...] * dinv


def _mid_body(p_ref, g0_ref, dinv_ref, g1_ref):
    a = p_ref[0, :N, :] + p_ref[1, :N, :] + g0_ref[...]
    di = dinv_ref[...]
    g1_ref[...] = a * (di * di)


def _final_body(q_ref, g1_ref, dinv_ref, b_ref, o_ref):
    h = (q_ref[0, :N, :] + q_ref[1, :N, :] + g1_ref[...]) * dinv_ref[...]
    h = h + b_ref[...]
    m = jnp.max(h, axis=1, keepdims=True)
    z = h - m
    o_ref[...] = z - jnp.log(jnp.sum(jnp.exp(z), axis=1, keepdims=True))


def kernel(x, edge_index, W, b):
    src = edge_index[0]
    dst = edge_index[1]
    pad = EP - E
    padk = jnp.arange(pad, dtype=jnp.int32)
    src_p = jnp.concatenate([src, padk % N])
    dst_p = jnp.concatenate([dst, NACC + (padk % NTRASH)])
    src3 = src_p.reshape(TOTCH, CHUNK)
    dst3 = dst_p.reshape(TOTCH, CHUNK)

    y = pl.pallas_call(
        _matmul_body,
        out_shape=jax.ShapeDtypeStruct((N, C), jnp.float32),
    )(x, W)

    degp = _deg_call(dst_p)

    g0, dinv = pl.pallas_call(
        _prep_body,
        out_shape=(jax.ShapeDtypeStruct((N, C), jnp.float32),
                   jax.ShapeDtypeStruct((N, 1), jnp.float32)),
    )(degp, y)

    p = _round_call(g0, src3, dst3)

    g1 = pl.pallas_call(
        _mid_body,
        out_shape=jax.ShapeDtypeStruct((N, C), jnp.float32),
    )(p, g0, dinv)

    q = _round_call(g1, src3, dst3)

    out = pl.pallas_call(
        _final_body,
        out_shape=jax.ShapeDtypeStruct((N, C), jnp.float32),
    )(q, g1, dinv, b)

    return out


# speedup vs baseline: 1.0573x; 1.0573x over previous
import dataclasses
import functools

import jax
import jax.numpy as jnp
from jax import lax
from jax.experimental import pallas as pl
from jax.experimental.pallas import tpu as pltpu
from jax.experimental.pallas import tpu_sc as plsc

N = 10000
D = 128
C = 64
E = 320000

NC = 2
NS = 16
NW = NC * NS
L = 16

CHUNK = 128
ITERS = 80
EPT = CHUNK * ITERS
EP = EPT * NW
TOTCH = EP // CHUNK
DOFF = E // CHUNK
IDXROWS = DOFF + TOTCH

CH0 = 80
CH1 = 80
CHMAX = max(CH0, CH1)

SLICE = 632
NACC = NS * SLICE
NTRASH = 128
ACC_ROWS = NACC + NTRASH
DEG_LEN = NACC + NTRASH

_mesh = plsc.VectorSubcoreMesh(core_axis_name="c", subcore_axis_name="s")

_sc_params = pltpu.CompilerParams(needs_layout_passes=False,
                                  use_tc_tiling_on_sc=False)


def _deg_body(idx_hbm, out_hbm, idx_v, deg_v):
    cid = lax.axis_index("c")
    sid = lax.axis_index("s")
    g = cid * NS + sid
    zero = jnp.zeros((L,), jnp.float32)

    @pl.loop(0, DEG_LEN // L)
    def _(i):
        deg_v[pl.ds(i * L, L)] = zero

    pltpu.sync_copy(idx_hbm.at[pl.ds(DOFF + g * ITERS, ITERS)], idx_v)
    ones = jnp.ones((L,), jnp.float32)

    @pl.loop(0, ITERS)
    def _(i):
        for k in range(CHUNK // L):
            idx = idx_v[i, pl.ds(k * L, L)]
            plsc.addupdate_scatter(deg_v, [idx], ones)

    pltpu.sync_copy(deg_v, out_hbm.at[g])


_deg_call = functools.partial(
    pl.kernel,
    out_type=jax.ShapeDtypeStruct((NW, DEG_LEN), jnp.float32),
    mesh=_mesh,
    compiler_params=_sc_params,
    scratch_types=[
        pltpu.VMEM((ITERS, CHUNK), jnp.int32),
        pltpu.VMEM((DEG_LEN,), jnp.float32),
    ],
)(_deg_body)


NBUF = 8


def _round_body(g_hbm, idx_hbm, out_hbm, src_v, dst_v,
                r0, r1, r2, r3, r4, r5, r6, r7, accum,
                gs0, gs1, gs2, gs3, gs4, gs5, gs6, gs7,
                ss0, ss1, ss2, ss3, ss4, ss5, ss6, ss7):
    rows = (r0, r1, r2, r3, r4, r5, r6, r7)
    gsem = (gs0, gs1, gs2, gs3, gs4, gs5, gs6, gs7)
    ssem = (ss0, ss1, ss2, ss3, ss4, ss5, ss6, ss7)
    cid = lax.axis_index("c")
    sid = lax.axis_index("s")
    zero = jnp.zeros((L,), jnp.float32)

    @pl.loop(0, CHUNK)
    def _(r):
        for k in range(C // L):
            r0[r, pl.ds(k * L, L)] = zero

    base = sid * SLICE
    nfull = SLICE // CHUNK
    rem = SLICE - nfull * CHUNK
    for k in range(nfull):
        pltpu.sync_copy(r0, accum.at[pl.ds(base + k * CHUNK, CHUNK)])
    if rem:
        pltpu.sync_copy(r0.at[pl.ds(0, rem)],
                        accum.at[pl.ds(base + nfull * CHUNK, rem)])
    plsc.subcore_barrier()

    def run_edges(start, cnt):
        pltpu.sync_copy(idx_hbm.at[pl.ds(start, cnt)],
                        src_v.at[pl.ds(0, cnt)])
        pltpu.sync_copy(idx_hbm.at[pl.ds(DOFF + start, cnt)],
                        dst_v.at[pl.ds(0, cnt)])

        for b in range(NBUF):
            pltpu.async_copy(g_hbm.at[src_v.at[b]], rows[b], gsem[b])

        @pl.loop(0, cnt, step=NBUF)
        def _(i0):
            for b in range(NBUF):
                i = i0 + b
                pltpu.make_async_copy(g_hbm.at[src_v.at[i]], rows[b],
                                      gsem[b]).wait()
                pltpu.async_copy(rows[b], accum.at[dst_v.at[i]], ssem[b],
                                 add=True)

                @pl.when(i0 + NBUF < cnt)
                def _():
                    pltpu.make_async_copy(rows[b], accum.at[dst_v.at[i]],
                                          ssem[b]).wait()
                    pltpu.async_copy(g_hbm.at[src_v.at[i + NBUF]], rows[b],
                                     gsem[b])

        for b in range(NBUF):
            i_last = cnt - NBUF + b
            pltpu.make_async_copy(rows[b], accum.at[dst_v.at[i_last]],
                                  ssem[b]).wait()

    if CH0:
        @pl.when(cid == 0)
        def _():
            run_edges(sid * CH0, CH0)

    if CH1:
        @pl.when(cid == 1)
        def _():
            run_edges(NS * CH0 + sid * CH1, CH1)

    plsc.subcore_barrier()
    pltpu.sync_copy(accum.at[pl.ds(sid * SLICE, SLICE)],
                    out_hbm.at[cid].at[pl.ds(sid * SLICE, SLICE)])


_round_call = functools.partial(
    pl.kernel,
    out_type=jax.ShapeDtypeStruct((NC, NACC, C), jnp.float32),
    mesh=_mesh,
    compiler_params=_sc_params,
    scratch_types=[
        pltpu.VMEM((CHMAX, CHUNK), jnp.int32),
        pltpu.VMEM((CHMAX, CHUNK), jnp.int32),
        pltpu.VMEM((CHUNK, C), jnp.float32),
        pltpu.VMEM((CHUNK, C), jnp.float32),
        pltpu.VMEM((CHUNK, C), jnp.float32),
        pltpu.VMEM((CHUNK, C), jnp.float32),
        pltpu.VMEM((CHUNK, C), jnp.float32),
        pltpu.VMEM((CHUNK, C), jnp.float32),
        pltpu.VMEM((CHUNK, C), jnp.float32),
        pltpu.VMEM((CHUNK, C), jnp.float32),
        pltpu.VMEM_SHARED((ACC_ROWS, C), jnp.float32),
        pltpu.SemaphoreType.DMA,
        pltpu.SemaphoreType.DMA,
        pltpu.SemaphoreType.DMA,
        pltpu.SemaphoreType.DMA,
        pltpu.SemaphoreType.DMA,
        pltpu.SemaphoreType.DMA,
        pltpu.SemaphoreType.DMA,
        pltpu.SemaphoreType.DMA,
        pltpu.SemaphoreType.DMA,
        pltpu.SemaphoreType.DMA,
        pltpu.SemaphoreType.DMA,
        pltpu.SemaphoreType.DMA,
        pltpu.SemaphoreType.DMA,
        pltpu.SemaphoreType.DMA,
        pltpu.SemaphoreType.DMA,
        pltpu.SemaphoreType.DMA,
    ],
)(_round_body)


def _matmul_body(x_ref, w_ref, y_ref):
    y_ref[...] = jnp.dot(x_ref[...], w_ref[...],
                         preferred_element_type=jnp.float32)


def _prep_body(degp_ref, y_ref, g0_ref, dinv_ref):
    deg = jnp.sum(degp_ref[...], axis=0)[:N] + 1.0
    dinv = lax.rsqrt(deg).reshape(N, 1)
    dinv_ref[...] = dinv
    g0_ref[...] = y_ref[...] * dinv


def _mid_body(p_ref, g0_ref, dinv_ref, g1_ref):
    a = p_ref[0, :N, :] + p_ref[1, :N, :] + g0_ref[...]
    di = dinv_ref[...]
    g1_ref[...] = a * (di * di)


def _final_body(q_ref, g1_ref, dinv_ref, b_ref, o_ref):
    h = (q_ref[0, :N, :] + q_ref[1, :N, :] + g1_ref[...]) * dinv_ref[...]
    h = h + b_ref[...]
    m = jnp.max(h, axis=1, keepdims=True)
    z = h - m
    o_ref[...] = z - jnp.log(jnp.sum(jnp.exp(z), axis=1, keepdims=True))


def kernel(x, edge_index, W, b):
    padk = jnp.arange(EP - E, dtype=jnp.int32)
    idx2 = jnp.concatenate(
        [edge_index.reshape(2 * E), NACC + (padk % NTRASH)]
    ).reshape(IDXROWS, CHUNK)

    y = pl.pallas_call(
        _matmul_body,
        out_shape=jax.ShapeDtypeStruct((N, C), jnp.float32),
    )(x, W)

    degp = _deg_call(idx2)

    g0, dinv = pl.pallas_call(
        _prep_body,
        out_shape=(jax.ShapeDtypeStruct((N, C), jnp.float32),
                   jax.ShapeDtypeStruct((N, 1), jnp.float32)),
    )(degp, y)

    p = _round_call(g0, idx2)

    g1 = pl.pallas_call(
        _mid_body,
        out_shape=jax.ShapeDtypeStruct((N, C), jnp.float32),
    )(p, g0, dinv)

    q = _round_call(g1, idx2)

    out = pl.pallas_call(
        _final_body,
        out_shape=jax.ShapeDtypeStruct((N, C), jnp.float32),
    )(q, g1, dinv, b)

    return out
